# Initial kernel scaffold; baseline (speedup 1.0000x reference)
import jax
import jax.numpy as jnp
from jax.experimental import pallas as pl


def kernel(predicted_boxes, predicted_labels, boxes, labels, default_boxes):
    raise NotImplementedError("write your pallas kernel here")


# speedup vs baseline: 8.1031x; 8.1031x over previous
import jax
import jax.numpy as jnp
from jax import lax
from jax.experimental import pallas as pl

_THRESHOLD = 0.5
_RATIO = 3.0
_ALPHA = 1.0

_DC = 2208
_NCH = 4


def _match_kernel(dbt_ref, boxes_ref, labels_ref, pbt_ref,
                  lab_ref, stats_ref):
    b = pl.program_id(0)
    nb = pl.num_programs(0)
    D = dbt_ref.shape[1]
    G = boxes_ref.shape[1]

    cx = dbt_ref[0:1, :]
    cy = dbt_ref[1:2, :]
    w = dbt_ref[2:3, :]
    h = dbt_ref[3:4, :]
    ux1 = cx - w / 2.0
    uy1 = cy - h / 2.0
    ux2 = w
    uy2 = h

    bx = boxes_ref[0]
    gx1 = bx[:, 0:1]
    gy1 = bx[:, 1:2]
    gx2 = bx[:, 2:3]
    gy2 = bx[:, 3:4]

    lbx = jnp.maximum(gx1, ux1)
    lby = jnp.maximum(gy1, uy1)
    ubx = jnp.minimum(gx2, ux2)
    uby = jnp.minimum(gy2, uy2)
    iw = jnp.clip(ubx - lbx, 0.0, None)
    ih = jnp.clip(uby - lby, 0.0, None)
    inter = iw * ih
    ar1 = (gx2 - gx1) * (gy2 - gy1)
    ar2 = (ux2 - ux1) * (uy2 - uy1)
    union = ar1 + ar2 - inter
    ious = inter / union

    gi = lax.broadcasted_iota(jnp.int32, (G, D), 0)
    di = lax.broadcasted_iota(jnp.int32, (G, D), 1)

    db_val = jnp.max(ious, axis=0, keepdims=True)
    db_box = jnp.min(jnp.where(ious == db_val, gi, G), axis=0,
                     keepdims=True)
    gmax = jnp.max(ious, axis=1, keepdims=True)
    box_db = jnp.min(jnp.where(ious == gmax, di, D), axis=1,
                     keepdims=True)

    eq = box_db == di
    g_last = jnp.max(jnp.where(eq, gi, -1), axis=0, keepdims=True)
    forced = g_last >= 0
    db_box = jnp.where(forced, g_last, db_box)
    db_val = jnp.where(forced, jnp.float32(_THRESHOLD), db_val)

    onehot = db_box == gi
    labels_col = labels_ref[0]
    lab = jnp.sum(jnp.where(onehot, labels_col, 0), axis=0,
                  keepdims=True)
    lab = jnp.where(db_val < _THRESHOLD, 0, lab)
    mask = (lab != 0).astype(jnp.float32)
    npos = jnp.sum(mask)

    s0 = jnp.sum(jnp.where(onehot, gx1, 0.0), axis=0, keepdims=True)
    s1 = jnp.sum(jnp.where(onehot, gy1, 0.0), axis=0, keepdims=True)
    s2 = jnp.sum(jnp.where(onehot, gx2, 0.0), axis=0, keepdims=True)
    s3 = jnp.sum(jnp.where(onehot, gy2, 0.0), axis=0, keepdims=True)
    rcx = (s0 + s2) / 2.0
    rcy = (s1 + s3) / 2.0
    gt0 = (rcx - cx) / w
    gt1 = (rcy - cy) / h
    gt2 = jnp.log(s2 / w)
    gt3 = jnp.log(s3 / h)

    pbt = pbt_ref[0]
    l1 = (jnp.abs(pbt[0:1, :] - gt0) + jnp.abs(pbt[1:2, :] - gt1)
          + jnp.abs(pbt[2:3, :] - gt2) + jnp.abs(pbt[3:4, :] - gt3))
    l1s = jnp.sum(l1 * mask)

    lab_ref[...] = lab.reshape(1, 1, D)

    ri8 = lax.broadcasted_iota(jnp.int32, (8, nb), 0)
    li8 = lax.broadcasted_iota(jnp.int32, (8, nb), 1)
    vals = jnp.where(ri8 == 0, npos, jnp.where(ri8 == 1, l1s, 0.0))
    stats_ref[...] = jnp.where(li8 == b, vals, stats_ref[...])


def _ce_kernel(plab_ref, lab_ref, neg_ref, stats_ref, *, d_total):
    c = pl.program_id(0)
    b = pl.program_id(1)
    nb = pl.num_programs(1)
    Dc, C = plab_ref.shape[1], plab_ref.shape[2]

    logits = plab_ref[0]
    lab = lab_ref[0][:, 0:1]
    row = lax.broadcasted_iota(jnp.int32, (Dc, 1), 0)
    row_ok = (c * Dc + row) < d_total

    m = jnp.max(logits, axis=1, keepdims=True)
    lse = jnp.log(jnp.sum(jnp.exp(logits - m), axis=1,
                          keepdims=True)) + m
    ci = lax.broadcasted_iota(jnp.int32, (Dc, C), 1)
    picked = jnp.sum(jnp.where(ci == lab, logits, 0.0), axis=1,
                     keepdims=True)
    closs = lse - picked
    mask = (lab != 0).astype(jnp.float32)
    pos_c = jnp.sum(jnp.where(row_ok, closs * mask, 0.0))
    neg_col = jnp.where(row_ok, closs * (1.0 - mask), 0.0)

    li = lax.broadcasted_iota(jnp.int32, (Dc, nb), 1)
    neg_ref[...] = jnp.where(li == b, neg_col, neg_ref[...])

    ri8 = lax.broadcasted_iota(jnp.int32, (8, nb), 0)
    li8 = lax.broadcasted_iota(jnp.int32, (8, nb), 1)
    prev = stats_ref[...]
    cur = jnp.where(c == 0, 0.0, prev)
    vals = jnp.where(ri8 == 0, pos_c, 0.0)
    stats_ref[...] = jnp.where(li8 == b, cur + vals, prev)


def _select_kernel(neg_ref, statsa_ref, statsb_ref, out_ref, *, d_total):
    v = neg_ref[...]
    vb = lax.bitcast_convert_type(v, jnp.int32)
    npos = statsa_ref[0:1, :]
    l1s = statsa_ref[1:2, :]
    poss = statsb_ref[0:1, :]
    k = jnp.minimum(npos * _RATIO, jnp.float32(d_total))

    t = jnp.zeros(npos.shape, jnp.int32)
    for bit in range(30, -1, -1):
        cand = t | (1 << bit)
        cnt = jnp.sum(jnp.where(vb >= cand, 1.0, 0.0), axis=0,
                      keepdims=True)
        t = jnp.where(cnt >= k, cand, t)
    tf = lax.bitcast_convert_type(t, jnp.float32)

    gtm = vb > t
    cnt_gt = jnp.sum(jnp.where(gtm, 1.0, 0.0), axis=0, keepdims=True)
    sum_gt = jnp.sum(jnp.where(gtm, v, 0.0), axis=0, keepdims=True)
    topk = sum_gt + (k - cnt_gt) * tf

    tot_np = jnp.sum(npos, keepdims=True)
    closs_total = (jnp.sum(topk, keepdims=True)
                   + jnp.sum(poss, keepdims=True)) / tot_np
    box_loss = jnp.sum(l1s, keepdims=True) / (tot_np * 4.0)
    out_ref[...] = closs_total + _ALPHA * box_loss


def kernel(predicted_boxes, predicted_labels, boxes, labels, default_boxes):
    import functools
    B, D, C = predicted_labels.shape
    G = boxes.shape[1]
    dpad = _DC * _NCH

    dbt = default_boxes.T
    pbt = jnp.transpose(predicted_boxes, (0, 2, 1))
    labels3 = labels.reshape(B, G, 1)

    lab, statsa = pl.pallas_call(
        _match_kernel,
        grid=(B,),
        in_specs=[
            pl.BlockSpec((4, D), lambda b: (0, 0)),
            pl.BlockSpec((1, G, 4), lambda b: (b, 0, 0)),
            pl.BlockSpec((1, G, 1), lambda b: (b, 0, 0)),
            pl.BlockSpec((1, 4, D), lambda b: (b, 0, 0)),
        ],
        out_specs=[
            pl.BlockSpec((1, 1, D), lambda b: (b, 0, 0)),
            pl.BlockSpec((8, B), lambda b: (0, 0)),
        ],
        out_shape=[
            jax.ShapeDtypeStruct((B, 1, D), jnp.int32),
            jax.ShapeDtypeStruct((8, B), jnp.float32),
        ],
    )(dbt, boxes, labels3, pbt)

    labt = lab.reshape(B, D, 1)

    neg, statsb = pl.pallas_call(
        functools.partial(_ce_kernel, d_total=D),
        grid=(_NCH, B),
        in_specs=[
            pl.BlockSpec((1, _DC, C), lambda c, b: (b, c, 0)),
            pl.BlockSpec((1, _DC, 1), lambda c, b: (b, c, 0)),
        ],
        out_specs=[
            pl.BlockSpec((_DC, B), lambda c, b: (c, 0)),
            pl.BlockSpec((8, B), lambda c, b: (0, 0)),
        ],
        out_shape=[
            jax.ShapeDtypeStruct((dpad, B), jnp.float32),
            jax.ShapeDtypeStruct((8, B), jnp.float32),
        ],
    )(predicted_labels, labt)

    out = pl.pallas_call(
        functools.partial(_select_kernel, d_total=D),
        in_specs=[
            pl.BlockSpec((dpad, B), lambda: (0, 0)),
            pl.BlockSpec((8, B), lambda: (0, 0)),
            pl.BlockSpec((8, B), lambda: (0, 0)),
        ],
        out_specs=pl.BlockSpec((1, 1), lambda: (0, 0)),
        out_shape=jax.ShapeDtypeStruct((1, 1), jnp.float32),
    )(neg, statsa, statsb)
    return out[0, 0]


# speedup vs baseline: 10.7917x; 1.3318x over previous
import jax
import jax.numpy as jnp
from jax import lax
from jax.experimental import pallas as pl

_THRESHOLD = 0.5
_RATIO = 3.0
_ALPHA = 1.0

_DC = 2944
_NCH = 3


def _match_kernel(dbt_ref, boxes_ref, labels_ref, pbt_ref,
                  lab_ref, stats_ref):
    b = pl.program_id(0)
    nb = pl.num_programs(0)
    D = dbt_ref.shape[1]
    G = boxes_ref.shape[1]

    cx = dbt_ref[0:1, :]
    cy = dbt_ref[1:2, :]
    w = dbt_ref[2:3, :]
    h = dbt_ref[3:4, :]
    ux1 = cx - w / 2.0
    uy1 = cy - h / 2.0
    ux2 = w
    uy2 = h

    bx = boxes_ref[0]
    gx1 = bx[:, 0:1]
    gy1 = bx[:, 1:2]
    gx2 = bx[:, 2:3]
    gy2 = bx[:, 3:4]

    lbx = jnp.maximum(gx1, ux1)
    lby = jnp.maximum(gy1, uy1)
    ubx = jnp.minimum(gx2, ux2)
    uby = jnp.minimum(gy2, uy2)
    iw = jnp.clip(ubx - lbx, 0.0, None)
    ih = jnp.clip(uby - lby, 0.0, None)
    inter = iw * ih
    ar1 = (gx2 - gx1) * (gy2 - gy1)
    ar2 = (ux2 - ux1) * (uy2 - uy1)
    union = ar1 + ar2 - inter
    ious = inter / union

    gi = lax.broadcasted_iota(jnp.int32, (G, D), 0)
    di = lax.broadcasted_iota(jnp.int32, (G, D), 1)

    db_val = jnp.max(ious, axis=0, keepdims=True)
    db_box = jnp.min(jnp.where(ious == db_val, gi, G), axis=0,
                     keepdims=True)
    gmax = jnp.max(ious, axis=1, keepdims=True)
    box_db = jnp.min(jnp.where(ious == gmax, di, D), axis=1,
                     keepdims=True)

    eq = box_db == di
    g_last = jnp.max(jnp.where(eq, gi, -1), axis=0, keepdims=True)
    forced = g_last >= 0
    db_box = jnp.where(forced, g_last, db_box)
    db_val = jnp.where(forced, jnp.float32(_THRESHOLD), db_val)

    onehot = db_box == gi
    labels_col = labels_ref[0]
    lab = jnp.sum(jnp.where(onehot, labels_col, 0), axis=0,
                  keepdims=True)
    lab = jnp.where(db_val < _THRESHOLD, 0, lab)
    mask = (lab != 0).astype(jnp.float32)
    npos = jnp.sum(mask)

    s0 = jnp.sum(jnp.where(onehot, gx1, 0.0), axis=0, keepdims=True)
    s1 = jnp.sum(jnp.where(onehot, gy1, 0.0), axis=0, keepdims=True)
    s2 = jnp.sum(jnp.where(onehot, gx2, 0.0), axis=0, keepdims=True)
    s3 = jnp.sum(jnp.where(onehot, gy2, 0.0), axis=0, keepdims=True)
    rcx = (s0 + s2) / 2.0
    rcy = (s1 + s3) / 2.0
    gt0 = (rcx - cx) / w
    gt1 = (rcy - cy) / h
    gt2 = jnp.log(s2 / w)
    gt3 = jnp.log(s3 / h)

    pbt = pbt_ref[0]
    l1 = (jnp.abs(pbt[0:1, :] - gt0) + jnp.abs(pbt[1:2, :] - gt1)
          + jnp.abs(pbt[2:3, :] - gt2) + jnp.abs(pbt[3:4, :] - gt3))
    l1s = jnp.sum(l1 * mask)

    lab_ref[...] = lab.reshape(1, 1, D)

    ri8 = lax.broadcasted_iota(jnp.int32, (8, nb), 0)
    li8 = lax.broadcasted_iota(jnp.int32, (8, nb), 1)
    vals = jnp.where(ri8 == 0, npos, jnp.where(ri8 == 1, l1s, 0.0))
    stats_ref[...] = jnp.where(li8 == b, vals, stats_ref[...])


def _ce_kernel(plab_ref, lab_ref, neg_ref, stats_ref, *, d_total):
    c = pl.program_id(0)
    b = pl.program_id(1)
    nb = pl.num_programs(1)
    Dc, C = plab_ref.shape[1], plab_ref.shape[2]

    logits = plab_ref[0]
    lab = jnp.transpose(lab_ref[0])
    row = lax.broadcasted_iota(jnp.int32, (Dc, 1), 0)
    row_ok = (c * Dc + row) < d_total

    ones = jnp.ones((C, 1), jnp.float32)
    m = jnp.max(logits, axis=1, keepdims=True)
    sexp = jnp.dot(jnp.exp(logits - m), ones,
                   preferred_element_type=jnp.float32)
    lse = jnp.log(sexp) + m
    ci = lax.broadcasted_iota(jnp.int32, (Dc, C), 1)
    picked = jnp.dot(jnp.where(ci == lab, logits, 0.0), ones,
                     preferred_element_type=jnp.float32)
    closs = lse - picked
    mask = (lab != 0).astype(jnp.float32)
    pos_c = jnp.sum(jnp.where(row_ok, closs * mask, 0.0))
    neg_col = jnp.where(row_ok, closs * (1.0 - mask), 0.0)

    li = lax.broadcasted_iota(jnp.int32, (Dc, nb), 1)
    neg_ref[...] = jnp.where(li == b, neg_col, neg_ref[...])

    ri8 = lax.broadcasted_iota(jnp.int32, (8, nb), 0)
    li8 = lax.broadcasted_iota(jnp.int32, (8, nb), 1)
    prev = stats_ref[...]
    cur = jnp.where(c == 0, 0.0, prev)
    vals = jnp.where(ri8 == 0, pos_c, 0.0)
    stats_ref[...] = jnp.where(li8 == b, cur + vals, prev)


def _select_kernel(neg_ref, statsa_ref, statsb_ref, out_ref, *, d_total):
    v = neg_ref[...]
    vb = lax.bitcast_convert_type(v, jnp.int32)
    npos = statsa_ref[0:1, :]
    l1s = statsa_ref[1:2, :]
    poss = statsb_ref[0:1, :]
    k = jnp.minimum(npos * _RATIO, jnp.float32(d_total))

    t = jnp.zeros(npos.shape, jnp.int32)
    for bit in range(30, -1, -1):
        cand = t | (1 << bit)
        cnt = jnp.sum(jnp.where(vb >= cand, 1.0, 0.0), axis=0,
                      keepdims=True)
        t = jnp.where(cnt >= k, cand, t)
    tf = lax.bitcast_convert_type(t, jnp.float32)

    gtm = vb > t
    cnt_gt = jnp.sum(jnp.where(gtm, 1.0, 0.0), axis=0, keepdims=True)
    sum_gt = jnp.sum(jnp.where(gtm, v, 0.0), axis=0, keepdims=True)
    topk = sum_gt + (k - cnt_gt) * tf

    tot_np = jnp.sum(npos, keepdims=True)
    closs_total = (jnp.sum(topk, keepdims=True)
                   + jnp.sum(poss, keepdims=True)) / tot_np
    box_loss = jnp.sum(l1s, keepdims=True) / (tot_np * 4.0)
    out_ref[...] = closs_total + _ALPHA * box_loss


def kernel(predicted_boxes, predicted_labels, boxes, labels, default_boxes):
    import functools
    B, D, C = predicted_labels.shape
    G = boxes.shape[1]
    dpad = _DC * _NCH

    dbt = default_boxes.T
    pbt = jnp.transpose(predicted_boxes, (0, 2, 1))
    labels3 = labels.reshape(B, G, 1)

    lab, statsa = pl.pallas_call(
        _match_kernel,
        grid=(B,),
        in_specs=[
            pl.BlockSpec((4, D), lambda b: (0, 0)),
            pl.BlockSpec((1, G, 4), lambda b: (b, 0, 0)),
            pl.BlockSpec((1, G, 1), lambda b: (b, 0, 0)),
            pl.BlockSpec((1, 4, D), lambda b: (b, 0, 0)),
        ],
        out_specs=[
            pl.BlockSpec((1, 1, D), lambda b: (b, 0, 0)),
            pl.BlockSpec((8, B), lambda b: (0, 0)),
        ],
        out_shape=[
            jax.ShapeDtypeStruct((B, 1, D), jnp.int32),
            jax.ShapeDtypeStruct((8, B), jnp.float32),
        ],
    )(dbt, boxes, labels3, pbt)

    neg, statsb = pl.pallas_call(
        functools.partial(_ce_kernel, d_total=D),
        grid=(_NCH, B),
        in_specs=[
            pl.BlockSpec((1, _DC, C), lambda c, b: (b, c, 0)),
            pl.BlockSpec((1, 1, _DC), lambda c, b: (b, 0, c)),
        ],
        out_specs=[
            pl.BlockSpec((_DC, B), lambda c, b: (c, 0)),
            pl.BlockSpec((8, B), lambda c, b: (0, 0)),
        ],
        out_shape=[
            jax.ShapeDtypeStruct((dpad, B), jnp.float32),
            jax.ShapeDtypeStruct((8, B), jnp.float32),
        ],
    )(predicted_labels, lab)

    out = pl.pallas_call(
        functools.partial(_select_kernel, d_total=D),
        in_specs=[
            pl.BlockSpec((dpad, B), lambda: (0, 0)),
            pl.BlockSpec((8, B), lambda: (0, 0)),
            pl.BlockSpec((8, B), lambda: (0, 0)),
        ],
        out_specs=pl.BlockSpec((1, 1), lambda: (0, 0)),
        out_shape=jax.ShapeDtypeStruct((1, 1), jnp.float32),
    )(neg, statsa, statsb)
    return out[0, 0]
